# baseline (speedup 1.0000x reference)
import functools

import jax
import jax.numpy as jnp
from jax.experimental import pallas as pl
from jax.experimental.pallas import tpu as pltpu

_BF = jnp.bfloat16
_G = 8


def _pool_bias_relu(conv, b_ref, scr_ref):
    m = conv.shape[0]
    scr_ref[...] = conv
    even = scr_ref[pl.ds(0, m // 2, 2), :]
    odd = scr_ref[pl.ds(1, m // 2, 2), :]
    return jnp.maximum(jnp.maximum(even, odd) + b_ref[...], 0.0)


def _shift3(a, g_ref, *, bblk, l):
    c = a.shape[-1]
    a3 = a.reshape(bblk, l, c)
    g_ref[:, 0:_G, :] = jnp.zeros((bblk, _G, c), jnp.float32)
    g_ref[:, l + _G:l + 2 * _G, :] = jnp.zeros((bblk, _G, c), jnp.float32)
    g_ref[:, _G:l + _G, :] = a3
    prev = g_ref[:, _G - 1:_G - 1 + l, :]
    nxt = g_ref[:, _G + 1:_G + 1 + l, :]
    return jnp.concatenate([prev, a3, nxt], axis=2).reshape(bblk * l, 3 * c)


def _fused_convs_kernel(x_ref, w1_ref, b1_ref, w2_ref, b2_ref, w3_ref, b3_ref,
                        o_ref, gy, g2, g3, p1, p2, *, bblk, seq):
    l2, l3 = seq // 2, seq // 4
    w1 = w1_ref[...]
    parts = []
    for b in range(bblk):
        parts.append(jax.lax.dot_general(
            x_ref[b].astype(_BF), w1, (((0,), (0,)), ((), ())),
            preferred_element_type=jnp.float32))
    y1 = jnp.concatenate(parts, axis=0) if bblk > 1 else parts[0]

    gy[:, 0:_G, :] = jnp.zeros((bblk, _G, 384), jnp.float32)
    gy[:, seq + _G:seq + 2 * _G, :] = jnp.zeros((bblk, _G, 384), jnp.float32)
    gy[:, _G:seq + _G, :] = y1.reshape(bblk, seq, 384)
    conv1 = (gy[:, _G:seq + _G, 128:256]
             + gy[:, _G - 1:_G - 1 + seq, 0:128]
             + gy[:, _G + 1:_G + 1 + seq, 256:384]).reshape(bblk * seq, 128)
    a1 = _pool_bias_relu(conv1, b1_ref, p1)

    x2 = _shift3(a1, g2, bblk=bblk, l=l2).astype(_BF)
    conv2 = jnp.dot(x2, w2_ref[...], preferred_element_type=jnp.float32)
    a2 = _pool_bias_relu(conv2, b2_ref, p2)

    x3 = _shift3(a2, g3, bblk=bblk, l=l3).astype(_BF)
    conv3 = jnp.dot(x3, w3_ref[...], preferred_element_type=jnp.float32)
    o_ref[...] = jnp.maximum(conv3 + b3_ref[...], 0.0).astype(_BF)


def _fc_head_kernel(a_ref, w1_ref, b1_ref, w2_ref, b2_ref, o_ref, acc_ref):
    @pl.when(pl.program_id(1) == 0)
    def _():
        acc_ref[...] = jnp.zeros_like(acc_ref)

    acc_ref[...] += jnp.dot(a_ref[...], w1_ref[...].astype(_BF),
                            preferred_element_type=jnp.float32)

    @pl.when(pl.program_id(1) == pl.num_programs(1) - 1)
    def _():
        h = jnp.maximum(acc_ref[...] + b1_ref[...], 0.0).astype(_BF)
        out = jnp.dot(h, w2_ref[...].astype(_BF),
                      preferred_element_type=jnp.float32)
        o_ref[...] = out + b2_ref[...]


def _taps_to_k(w_cat):
    return jnp.concatenate(
        [w_cat[:, 0:128], w_cat[:, 128:256], w_cat[:, 256:384]], axis=0)


def kernel(x, c1_w, c1_b, c2_w, c2_b, c3_w, c3_b, f1_w, f1_b, f2_w, f2_b):
    batch, cin, seq = x.shape
    l4 = seq // 4
    bblk = next(d for d in (8, 4, 2, 1) if batch % d == 0)
    m_out = bblk * l4

    w1 = c1_w.astype(_BF)
    w2k = _taps_to_k(c2_w).astype(_BF)
    w3k = _taps_to_k(c3_w).astype(_BF)

    body = functools.partial(_fused_convs_kernel, bblk=bblk, seq=seq)
    act = pl.pallas_call(
        body,
        out_shape=jax.ShapeDtypeStruct((batch * l4, 128), _BF),
        grid=(batch // bblk,),
        in_specs=[
            pl.BlockSpec((bblk, cin, seq), lambda i: (i, 0, 0)),
            pl.BlockSpec((cin, 384), lambda i: (0, 0)),
            pl.BlockSpec((1, 128), lambda i: (0, 0)),
            pl.BlockSpec((384, 128), lambda i: (0, 0)),
            pl.BlockSpec((1, 128), lambda i: (0, 0)),
            pl.BlockSpec((384, 128), lambda i: (0, 0)),
            pl.BlockSpec((1, 128), lambda i: (0, 0)),
        ],
        out_specs=pl.BlockSpec((m_out, 128), lambda i: (i, 0)),
        scratch_shapes=[
            pltpu.VMEM((bblk, seq + 2 * _G, 384), jnp.float32),
            pltpu.VMEM((bblk, seq // 2 + 2 * _G, 128), jnp.float32),
            pltpu.VMEM((bblk, l4 + 2 * _G, 128), jnp.float32),
            pltpu.VMEM((bblk * seq, 128), jnp.float32),
            pltpu.VMEM((bblk * (seq // 2), 128), jnp.float32),
        ],
        compiler_params=pltpu.CompilerParams(
            dimension_semantics=("parallel",)),
    )(x, w1, c1_b, w2k, c2_b, w3k, c3_b)

    a = act.reshape(batch, l4 * 128)
    k_tot = l4 * 128
    tk = min(8192, k_tot)
    bm = batch // 2 if batch % 2 == 0 else batch
    out = pl.pallas_call(
        _fc_head_kernel,
        out_shape=jax.ShapeDtypeStruct((batch, 128), jnp.float32),
        grid=(batch // bm, k_tot // tk),
        in_specs=[
            pl.BlockSpec((bm, tk), lambda i, ki: (i, ki)),
            pl.BlockSpec((tk, 128), lambda i, ki: (ki, 0)),
            pl.BlockSpec((1, 128), lambda i, ki: (0, 0)),
            pl.BlockSpec((128, 128), lambda i, ki: (0, 0)),
            pl.BlockSpec((1, 128), lambda i, ki: (0, 0)),
        ],
        out_specs=pl.BlockSpec((bm, 128), lambda i, ki: (i, 0)),
        scratch_shapes=[pltpu.VMEM((bm, 128), jnp.float32)],
        compiler_params=pltpu.CompilerParams(
            dimension_semantics=("parallel", "arbitrary")),
    )(a, f1_w, f1_b, f2_w, f2_b)
    return out[:, :10]
